# baseline (speedup 1.0000x reference)
import functools

import jax
import jax.numpy as jnp
from jax import lax
from jax.experimental import pallas as pl
from jax.experimental.pallas import tpu as pltpu
from jax.experimental.pallas import tpu_sc as plsc

T = 2048
D = 768
E = 8
FF = 2048
R = 256
NBR = 23
SH_OFF = NBR * R
NB = NBR + 8 + 1
ROWS = NB * R
NT = NBR + 8

_NC = 2
_NW = 32


def _gate_kernel(x_ref, gw_ref, pos_ref, w_ref, cnt_ref):
    x = x_ref[...]
    gw = gw_ref[...]
    logits = lax.dot_general(x, gw, (((1,), (1,)), ((), ())),
                             preferred_element_type=jnp.float32)
    m = jnp.max(logits, axis=-1, keepdims=True)
    p = jnp.exp(logits - m)
    s = p / jnp.sum(p, axis=-1, keepdims=True)
    ie = lax.broadcasted_iota(jnp.int32, (T, E), 1)
    m1 = jnp.max(s, axis=-1, keepdims=True)
    e0 = jnp.min(jnp.where(s == m1, ie, E), axis=-1, keepdims=True)
    s2 = jnp.where(ie == e0, -jnp.inf, s)
    m2 = jnp.max(s2, axis=-1, keepdims=True)
    e1 = jnp.min(jnp.where(s2 == m2, ie, E), axis=-1, keepdims=True)
    denom = m1 + m2 + 1e-20
    w_ref[:, 0:1] = m1 / denom
    w_ref[:, 1:2] = m2 / denom

    oh0 = (ie == e0).astype(jnp.float32)
    oh1 = (ie == e1).astype(jnp.float32)
    oh = jnp.concatenate([oh0, oh1], axis=0)
    CH = 512
    ci = lax.broadcasted_iota(jnp.int32, (CH, CH), 0)
    cj = lax.broadcasted_iota(jnp.int32, (CH, CH), 1)
    tri = (cj <= ci).astype(jnp.float32)
    carry = jnp.zeros((1, E), jnp.float32)
    chunks = []
    for i in range(2 * T // CH):
        ch = oh[i * CH:(i + 1) * CH]
        cs = lax.dot_general(tri, ch, (((1,), (0,)), ((), ())),
                             preferred_element_type=jnp.float32) + carry
        carry = cs[CH - 1:CH, :]
        chunks.append(cs)
    csum = jnp.concatenate(chunks, axis=0)
    cnt = carry
    nbf = jnp.floor((cnt + (R - 1)) / R)
    ui = lax.broadcasted_iota(jnp.int32, (E, E), 0)
    uj = lax.broadcasted_iota(jnp.int32, (E, E), 1)
    upper = (ui < uj).astype(jnp.float32)
    po = lax.dot_general(nbf, upper, (((1,), (0,)), ((), ())),
                         preferred_element_type=jnp.float32) * R
    po_sel = jnp.sum(oh * po, axis=-1, keepdims=True)
    rank = jnp.sum(oh * csum, axis=-1, keepdims=True) - 1.0
    pos_ref[...] = (po_sel + rank).astype(jnp.int32)
    cnt_ref[...] = cnt.astype(jnp.int32)


def _gate(xt, gate_w):
    return pl.pallas_call(
        _gate_kernel,
        out_shape=(
            jax.ShapeDtypeStruct((2 * T, 1), jnp.int32),
            jax.ShapeDtypeStruct((T, 2), jnp.float32),
            jax.ShapeDtypeStruct((1, E), jnp.int32),
        ),
    )(xt, gate_w)


def _dispatch_kernel(x_hbm, pos_hbm, xs_hbm, idx_v, rows_v, sem):
    wid = lax.axis_index("s") * _NC + lax.axis_index("c")
    base = wid * (3 * T // _NW)
    for c in range(3):
        g = base + c * 64
        pltpu.sync_copy(pos_hbm.at[pl.ds(g, 64)], idx_v.at[c])
        xrow = lax.rem(g, T)
        pltpu.sync_copy(x_hbm.at[pl.ds(xrow, 64)], rows_v)
        pltpu.async_copy(rows_v, xs_hbm.at[idx_v.at[c]], sem).wait()


def _dispatch(xt, pos_ext):
    mesh = plsc.VectorSubcoreMesh(core_axis_name="c", subcore_axis_name="s")
    return pl.kernel(
        _dispatch_kernel,
        out_type=jax.ShapeDtypeStruct((ROWS, D), jnp.float32),
        mesh=mesh,
        scratch_types=[
            pltpu.VMEM((3, 64), jnp.int32),
            pltpu.VMEM((64, D), jnp.float32),
            pltpu.SemaphoreType.DMA,
        ],
    )(xt, pos_ext)


def _ffn_kernel(xsb_ref, eid_ref, ob_ref, xs_ref, wg_ref, wu_ref, wd_ref,
                out_ref):
    del xsb_ref, eid_ref, ob_ref
    x = xs_ref[...]
    g = lax.dot_general(x, wg_ref[0], (((1,), (1,)), ((), ())),
                        preferred_element_type=jnp.float32)
    u = lax.dot_general(x, wu_ref[0], (((1,), (1,)), ((), ())),
                        preferred_element_type=jnp.float32)
    h = g * lax.logistic(g) * u
    out_ref[...] = lax.dot_general(h, wd_ref[0], (((1,), (1,)), ((), ())),
                                   preferred_element_type=jnp.float32)


def _gmm(xs, wg_cat, wu_cat, wd_cat, xs_blk, w_eid, out_blk):
    grid_spec = pltpu.PrefetchScalarGridSpec(
        num_scalar_prefetch=3,
        grid=(NT,),
        in_specs=[
            pl.BlockSpec((R, D), lambda t, xsb, eid, ob: (xsb[t], 0)),
            pl.BlockSpec((1, FF, D), lambda t, xsb, eid, ob: (eid[t], 0, 0)),
            pl.BlockSpec((1, FF, D), lambda t, xsb, eid, ob: (eid[t], 0, 0)),
            pl.BlockSpec((1, D, FF), lambda t, xsb, eid, ob: (eid[t], 0, 0)),
        ],
        out_specs=pl.BlockSpec((R, D), lambda t, xsb, eid, ob: (ob[t], 0)),
    )
    return pl.pallas_call(
        _ffn_kernel,
        grid_spec=grid_spec,
        out_shape=jax.ShapeDtypeStruct((ROWS, D), jnp.float32),
        compiler_params=pltpu.CompilerParams(
            dimension_semantics=("arbitrary",)),
    )(xs_blk, w_eid, out_blk, xs, wg_cat, wu_cat, wd_cat)


def _collect_kernel(out_hbm, pos_hbm, g_hbm, idx_v, rows_v, sem):
    wid = lax.axis_index("s") * _NC + lax.axis_index("c")
    base = wid * (2 * T // _NW)
    for c in range(2):
        g = base + c * 64
        pltpu.sync_copy(pos_hbm.at[pl.ds(g, 64)], idx_v.at[c])
        pltpu.async_copy(out_hbm.at[idx_v.at[c]], rows_v, sem).wait()
        pltpu.sync_copy(rows_v, g_hbm.at[pl.ds(g, 64)])


def _collect(out, pos_ext):
    mesh = plsc.VectorSubcoreMesh(core_axis_name="c", subcore_axis_name="s")
    return pl.kernel(
        _collect_kernel,
        out_type=jax.ShapeDtypeStruct((2 * T, D), jnp.float32),
        mesh=mesh,
        scratch_types=[
            pltpu.VMEM((2, 64), jnp.int32),
            pltpu.VMEM((64, D), jnp.float32),
            pltpu.SemaphoreType.DMA,
        ],
    )(out, pos_ext)


def _combine_kernel(g0_ref, g1_ref, sh_ref, w_ref, y_ref):
    b = pl.program_id(0)
    w0 = w_ref[pl.ds(b * R, R), 0:1]
    w1 = w_ref[pl.ds(b * R, R), 1:2]
    y_ref[...] = w0 * g0_ref[...] + w1 * g1_ref[...] + sh_ref[...]


def _combine(gath, out, tw):
    return pl.pallas_call(
        _combine_kernel,
        grid=(T // R,),
        in_specs=[
            pl.BlockSpec((R, D), lambda b: (b, 0)),
            pl.BlockSpec((R, D), lambda b: (b + T // R, 0)),
            pl.BlockSpec((R, D), lambda b: (b + NBR, 0)),
            pl.BlockSpec((T, 2), lambda b: (0, 0)),
        ],
        out_specs=pl.BlockSpec((R, D), lambda b: (b, 0)),
        out_shape=jax.ShapeDtypeStruct((T, D), jnp.float32),
    )(gath, gath, out, tw)


def kernel(x, gate_w, w_gate, w_up, w_down, sw_gate, sw_up, sw_down):
    b, s, h = x.shape
    xt = x.reshape(-1, h)

    pos, tw, cnt = _gate(xt, gate_w)

    c = cnt[0]
    nb = (c + R - 1) // R
    starts = jnp.cumsum(nb)
    nbr = starts[E - 1]
    ti = jnp.arange(NBR, dtype=jnp.int32)
    eid_r = jnp.minimum(
        jnp.sum((starts[None, :] <= ti[:, None]).astype(jnp.int32), axis=1),
        E - 1)
    sh_ids = jnp.arange(8, dtype=jnp.int32)
    w_eid = jnp.concatenate([eid_r, jnp.full((8,), E, jnp.int32)])
    xs_blk = jnp.concatenate([jnp.minimum(ti, nbr - 1), NBR + sh_ids])
    out_blk = jnp.concatenate(
        [jnp.where(ti < nbr, ti, NB - 1), NBR + sh_ids])

    pos_ext = jnp.concatenate(
        [pos.reshape(2 * T), SH_OFF + jnp.arange(T, dtype=jnp.int32)])

    xs = _dispatch(xt, pos_ext)

    wg_cat = jnp.concatenate([w_gate, sw_gate[None]], axis=0)
    wu_cat = jnp.concatenate([w_up, sw_up[None]], axis=0)
    wd_cat = jnp.concatenate([w_down, sw_down[None]], axis=0)
    out = _gmm(xs, wg_cat, wu_cat, wd_cat, xs_blk, w_eid, out_blk)

    gath = _collect(out, pos_ext)
    y = _combine(gath, out, tw)
    return y.reshape(b, s, h)


# speedup vs baseline: 1.1382x; 1.1382x over previous
import functools

import jax
import jax.numpy as jnp
from jax import lax
from jax.experimental import pallas as pl
from jax.experimental.pallas import tpu as pltpu
from jax.experimental.pallas import tpu_sc as plsc

T = 2048
D = 768
E = 8
FF = 2048
R = 256
NBR = 23
SH_OFF = NBR * R
NB = NBR + 8 + 1
ROWS = NB * R
NT = NBR + 8

_NC = 2
_NW = 32


def _gate_kernel(x_ref, gw_ref, pos_ref, w_ref, cnt_ref):
    x = x_ref[...]
    gw = gw_ref[...]
    logits = lax.dot_general(x, gw, (((1,), (1,)), ((), ())),
                             preferred_element_type=jnp.float32)
    m = jnp.max(logits, axis=-1, keepdims=True)
    p = jnp.exp(logits - m)
    s = p / jnp.sum(p, axis=-1, keepdims=True)
    ie = lax.broadcasted_iota(jnp.int32, (T, E), 1)
    m1 = jnp.max(s, axis=-1, keepdims=True)
    e0 = jnp.min(jnp.where(s == m1, ie, E), axis=-1, keepdims=True)
    s2 = jnp.where(ie == e0, -jnp.inf, s)
    m2 = jnp.max(s2, axis=-1, keepdims=True)
    e1 = jnp.min(jnp.where(s2 == m2, ie, E), axis=-1, keepdims=True)
    denom = m1 + m2 + 1e-20
    w_ref[:, 0:1] = m1 / denom
    w_ref[:, 1:2] = m2 / denom

    oh0 = (ie == e0).astype(jnp.float32)
    oh1 = (ie == e1).astype(jnp.float32)
    oh = jnp.concatenate([oh0, oh1], axis=0)
    CH = 512
    ci = lax.broadcasted_iota(jnp.int32, (CH, CH), 0)
    cj = lax.broadcasted_iota(jnp.int32, (CH, CH), 1)
    tri = (cj <= ci).astype(jnp.float32)
    carry = jnp.zeros((1, E), jnp.float32)
    chunks = []
    for i in range(2 * T // CH):
        ch = oh[i * CH:(i + 1) * CH]
        cs = lax.dot_general(tri, ch, (((1,), (0,)), ((), ())),
                             preferred_element_type=jnp.float32) + carry
        carry = cs[CH - 1:CH, :]
        chunks.append(cs)
    csum = jnp.concatenate(chunks, axis=0)
    cnt = carry
    nbf = jnp.floor((cnt + (R - 1)) / R)
    ui = lax.broadcasted_iota(jnp.int32, (E, E), 0)
    uj = lax.broadcasted_iota(jnp.int32, (E, E), 1)
    upper = (ui < uj).astype(jnp.float32)
    po = lax.dot_general(nbf, upper, (((1,), (0,)), ((), ())),
                         preferred_element_type=jnp.float32) * R
    po_sel = jnp.sum(oh * po, axis=-1, keepdims=True)
    rank = jnp.sum(oh * csum, axis=-1, keepdims=True) - 1.0
    pos_ref[...] = (po_sel + rank).astype(jnp.int32)
    cnt_ref[...] = cnt.astype(jnp.int32)


def _gate(xt, gate_w):
    return pl.pallas_call(
        _gate_kernel,
        out_shape=(
            jax.ShapeDtypeStruct((2 * T, 1), jnp.int32),
            jax.ShapeDtypeStruct((T, 2), jnp.float32),
            jax.ShapeDtypeStruct((1, E), jnp.int32),
        ),
    )(xt, gate_w)


def _dispatch_kernel(x_hbm, pos_hbm, xs_hbm, idx_v, rows_v, sem):
    wid = lax.axis_index("s") * _NC + lax.axis_index("c")
    base = wid * (3 * T // _NW)
    for c in range(3):
        g = base + c * 64
        pltpu.sync_copy(pos_hbm.at[pl.ds(g, 64)], idx_v.at[c])
        xrow = lax.rem(g, T)
        pltpu.sync_copy(x_hbm.at[pl.ds(xrow, 64)], rows_v)
        pltpu.async_copy(rows_v, xs_hbm.at[idx_v.at[c]], sem).wait()


def _dispatch(xt, pos_ext):
    mesh = plsc.VectorSubcoreMesh(core_axis_name="c", subcore_axis_name="s")
    return pl.kernel(
        _dispatch_kernel,
        out_type=jax.ShapeDtypeStruct((ROWS, D), jnp.float32),
        mesh=mesh,
        scratch_types=[
            pltpu.VMEM((3, 64), jnp.int32),
            pltpu.VMEM((64, D), jnp.float32),
            pltpu.SemaphoreType.DMA,
        ],
    )(xt, pos_ext)


FH = FF // 2


def _ffn_block(x, wg, wu, wd, out_ref, first):
    xb = x.astype(jnp.bfloat16)
    g = lax.dot_general(xb, wg.astype(jnp.bfloat16), (((1,), (1,)), ((), ())),
                        preferred_element_type=jnp.float32)
    u = lax.dot_general(xb, wu.astype(jnp.bfloat16), (((1,), (1,)), ((), ())),
                        preferred_element_type=jnp.float32)
    h = (g * lax.logistic(g) * u).astype(jnp.bfloat16)
    part = lax.dot_general(h, wd.astype(jnp.bfloat16), (((1,), (1,)), ((), ())),
                           preferred_element_type=jnp.float32)

    @pl.when(first)
    def _():
        out_ref[...] = part

    @pl.when(jnp.logical_not(first))
    def _():
        out_ref[...] = out_ref[...] + part


def _ffn_kernel(xsb_ref, eid_ref, ob_ref, kd_ref, xs_ref, wg_ref, wu_ref,
                wd_ref, swg_ref, swu_ref, swd_ref, out_ref):
    del xsb_ref, eid_ref, ob_ref
    t = pl.program_id(0)
    f = pl.program_id(1)
    x = xs_ref[...]

    @pl.when(kd_ref[t] == 0)
    def _():
        _ffn_block(x, wg_ref[0], wu_ref[0], wd_ref[0], out_ref, f == 0)

    @pl.when(kd_ref[t] != 0)
    def _():
        fs = pl.ds(f * FH, FH)
        _ffn_block(x, swg_ref[fs, :], swu_ref[fs, :], swd_ref[:, fs],
                   out_ref, f == 0)


def _gmm(xs, w_gate, w_up, w_down, sw_gate, sw_up, sw_down,
         xs_blk, w_eid, out_blk, kind):
    grid_spec = pltpu.PrefetchScalarGridSpec(
        num_scalar_prefetch=4,
        grid=(NT, 2),
        in_specs=[
            pl.BlockSpec((R, D), lambda t, f, xsb, eid, ob, kd: (xsb[t], 0)),
            pl.BlockSpec((1, FH, D),
                         lambda t, f, xsb, eid, ob, kd: (eid[t], f, 0)),
            pl.BlockSpec((1, FH, D),
                         lambda t, f, xsb, eid, ob, kd: (eid[t], f, 0)),
            pl.BlockSpec((1, D, FH),
                         lambda t, f, xsb, eid, ob, kd: (eid[t], 0, f)),
            pl.BlockSpec((FF, D), lambda t, f, xsb, eid, ob, kd: (0, 0)),
            pl.BlockSpec((FF, D), lambda t, f, xsb, eid, ob, kd: (0, 0)),
            pl.BlockSpec((D, FF), lambda t, f, xsb, eid, ob, kd: (0, 0)),
        ],
        out_specs=pl.BlockSpec((R, D), lambda t, f, xsb, eid, ob, kd: (ob[t], 0)),
    )
    return pl.pallas_call(
        _ffn_kernel,
        grid_spec=grid_spec,
        out_shape=jax.ShapeDtypeStruct((ROWS, D), jnp.float32),
        compiler_params=pltpu.CompilerParams(
            dimension_semantics=("arbitrary", "arbitrary")),
    )(xs_blk, w_eid, out_blk, kind, xs, w_gate, w_up, w_down,
      sw_gate, sw_up, sw_down)


def _collect_kernel(out_hbm, pos_hbm, g_hbm, idx_v, rows_v, sem):
    wid = lax.axis_index("s") * _NC + lax.axis_index("c")
    base = wid * (2 * T // _NW)
    for c in range(2):
        g = base + c * 64
        pltpu.sync_copy(pos_hbm.at[pl.ds(g, 64)], idx_v.at[c])
        pltpu.async_copy(out_hbm.at[idx_v.at[c]], rows_v, sem).wait()
        pltpu.sync_copy(rows_v, g_hbm.at[pl.ds(g, 64)])


def _collect(out, pos_ext):
    mesh = plsc.VectorSubcoreMesh(core_axis_name="c", subcore_axis_name="s")
    return pl.kernel(
        _collect_kernel,
        out_type=jax.ShapeDtypeStruct((2 * T, D), jnp.float32),
        mesh=mesh,
        scratch_types=[
            pltpu.VMEM((2, 64), jnp.int32),
            pltpu.VMEM((64, D), jnp.float32),
            pltpu.SemaphoreType.DMA,
        ],
    )(out, pos_ext)


def _combine_kernel(g0_ref, g1_ref, sh_ref, w_ref, y_ref):
    b = pl.program_id(0)
    w0 = w_ref[pl.ds(b * R, R), 0:1]
    w1 = w_ref[pl.ds(b * R, R), 1:2]
    y_ref[...] = w0 * g0_ref[...] + w1 * g1_ref[...] + sh_ref[...]


def _combine(gath, out, tw):
    return pl.pallas_call(
        _combine_kernel,
        grid=(T // R,),
        in_specs=[
            pl.BlockSpec((R, D), lambda b: (b, 0)),
            pl.BlockSpec((R, D), lambda b: (b + T // R, 0)),
            pl.BlockSpec((R, D), lambda b: (b + NBR, 0)),
            pl.BlockSpec((T, 2), lambda b: (0, 0)),
        ],
        out_specs=pl.BlockSpec((R, D), lambda b: (b, 0)),
        out_shape=jax.ShapeDtypeStruct((T, D), jnp.float32),
    )(gath, gath, out, tw)


def kernel(x, gate_w, w_gate, w_up, w_down, sw_gate, sw_up, sw_down):
    b, s, h = x.shape
    xt = x.reshape(-1, h)

    pos, tw, cnt = _gate(xt, gate_w)

    c = cnt[0]
    nb = (c + R - 1) // R
    starts = jnp.cumsum(nb)
    nbr = starts[E - 1]
    ti = jnp.arange(NBR, dtype=jnp.int32)
    eid_r = jnp.minimum(
        jnp.sum((starts[None, :] <= ti[:, None]).astype(jnp.int32), axis=1),
        E - 1)
    sh_ids = jnp.arange(8, dtype=jnp.int32)
    w_eid = jnp.concatenate([eid_r, jnp.full((8,), eid_r[-1], jnp.int32)])
    xs_blk = jnp.concatenate([jnp.minimum(ti, nbr - 1), NBR + sh_ids])
    out_blk = jnp.concatenate(
        [jnp.where(ti < nbr, ti, NB - 1), NBR + sh_ids])
    kind = jnp.concatenate(
        [jnp.zeros((NBR,), jnp.int32), jnp.ones((8,), jnp.int32)])

    pos_ext = jnp.concatenate(
        [pos.reshape(2 * T), SH_OFF + jnp.arange(T, dtype=jnp.int32)])

    xs = _dispatch(xt, pos_ext)

    out = _gmm(xs, w_gate, w_up, w_down, sw_gate, sw_up, sw_down,
               xs_blk, w_eid, out_blk, kind)

    gath = _collect(out, pos_ext)
    y = _combine(gath, out, tw)
    return y.reshape(b, s, h)


# speedup vs baseline: 1.5086x; 1.3254x over previous
import jax
import jax.numpy as jnp
from jax import lax
from jax.experimental import pallas as pl
from jax.experimental.pallas import tpu as pltpu
from jax.experimental.pallas import tpu_sc as plsc

T = 2048
D = 768
E = 8
FF = 2048
R = 256
NBR = 23
NB = NBR + 1
ROWS = NB * R

_NC = 2
_NW = 32


def _gate_kernel(x_ref, gw_ref, pos_ref, w_ref, cnt_ref):
    x = x_ref[...]
    gw = gw_ref[...]
    logits = lax.dot_general(x, gw, (((1,), (1,)), ((), ())),
                             preferred_element_type=jnp.float32)
    m = jnp.max(logits, axis=-1, keepdims=True)
    p = jnp.exp(logits - m)
    s = p / jnp.sum(p, axis=-1, keepdims=True)
    ie = lax.broadcasted_iota(jnp.int32, (T, E), 1)
    m1 = jnp.max(s, axis=-1, keepdims=True)
    e0 = jnp.min(jnp.where(s == m1, ie, E), axis=-1, keepdims=True)
    s2 = jnp.where(ie == e0, -jnp.inf, s)
    m2 = jnp.max(s2, axis=-1, keepdims=True)
    e1 = jnp.min(jnp.where(s2 == m2, ie, E), axis=-1, keepdims=True)
    denom = m1 + m2 + 1e-20
    w_ref[:, 0:1] = m1 / denom
    w_ref[:, 1:2] = m2 / denom

    oh0 = (ie == e0).astype(jnp.float32)
    oh1 = (ie == e1).astype(jnp.float32)
    oh = jnp.concatenate([oh0, oh1], axis=0)
    CH = 512
    ci = lax.broadcasted_iota(jnp.int32, (CH, CH), 0)
    cj = lax.broadcasted_iota(jnp.int32, (CH, CH), 1)
    tri = (cj <= ci).astype(jnp.float32)
    carry = jnp.zeros((1, E), jnp.float32)
    chunks = []
    for i in range(2 * T // CH):
        ch = oh[i * CH:(i + 1) * CH]
        cs = lax.dot_general(tri, ch, (((1,), (0,)), ((), ())),
                             preferred_element_type=jnp.float32) + carry
        carry = cs[CH - 1:CH, :]
        chunks.append(cs)
    csum = jnp.concatenate(chunks, axis=0)
    cnt = carry
    nbf = jnp.floor((cnt + (R - 1)) / R)
    ui = lax.broadcasted_iota(jnp.int32, (E, E), 0)
    uj = lax.broadcasted_iota(jnp.int32, (E, E), 1)
    upper = (ui < uj).astype(jnp.float32)
    po = lax.dot_general(nbf, upper, (((1,), (0,)), ((), ())),
                         preferred_element_type=jnp.float32) * R
    po_sel = jnp.sum(oh * po, axis=-1, keepdims=True)
    rank = jnp.sum(oh * csum, axis=-1, keepdims=True) - 1.0
    pos_ref[...] = (po_sel + rank).astype(jnp.int32)
    cnt_ref[...] = cnt.astype(jnp.int32)


def _gate(xt, gate_w):
    return pl.pallas_call(
        _gate_kernel,
        out_shape=(
            jax.ShapeDtypeStruct((2 * T, 1), jnp.int32),
            jax.ShapeDtypeStruct((T, 2), jnp.float32),
            jax.ShapeDtypeStruct((1, E), jnp.int32),
        ),
    )(xt, gate_w)


def _dispatch_kernel(x_hbm, pos_hbm, xs_hbm, idx_v, rows_v, sem):
    wid = lax.axis_index("s") * _NC + lax.axis_index("c")
    base = wid * (2 * T // _NW)
    for c in range(2):
        g = base + c * 64
        pltpu.sync_copy(pos_hbm.at[pl.ds(g, 64)], idx_v.at[c])
        xrow = lax.rem(g, T)
        pltpu.sync_copy(x_hbm.at[pl.ds(xrow, 64)], rows_v)
        pltpu.async_copy(rows_v, xs_hbm.at[idx_v.at[c]], sem).wait()


def _dispatch(xt, pos):
    mesh = plsc.VectorSubcoreMesh(core_axis_name="c", subcore_axis_name="s")
    return pl.kernel(
        _dispatch_kernel,
        out_type=jax.ShapeDtypeStruct((ROWS, D), jnp.float32),
        mesh=mesh,
        scratch_types=[
            pltpu.VMEM((2, 64), jnp.int32),
            pltpu.VMEM((64, D), jnp.float32),
            pltpu.SemaphoreType.DMA,
        ],
    )(xt, pos)


def _shared_kernel(x_ref, swg_ref, swu_ref, swd_ref, out_ref,
                   bg_ref, bu_ref, bd_ref):
    b = pl.program_id(0)

    @pl.when(b == 0)
    def _():
        bg_ref[...] = swg_ref[...].astype(jnp.bfloat16)
        bu_ref[...] = swu_ref[...].astype(jnp.bfloat16)
        bd_ref[...] = swd_ref[...].astype(jnp.bfloat16)

    xb = x_ref[...].astype(jnp.bfloat16)
    g = lax.dot_general(xb, bg_ref[...], (((1,), (1,)), ((), ())),
                        preferred_element_type=jnp.float32)
    u = lax.dot_general(xb, bu_ref[...], (((1,), (1,)), ((), ())),
                        preferred_element_type=jnp.float32)
    h = (g * lax.logistic(g) * u).astype(jnp.bfloat16)
    out_ref[...] = lax.dot_general(h, bd_ref[...], (((1,), (1,)), ((), ())),
                                   preferred_element_type=jnp.float32)


def _shared_ffn(xt, sw_gate, sw_up, sw_down):
    return pl.pallas_call(
        _shared_kernel,
        grid=(T // R,),
        in_specs=[
            pl.BlockSpec((R, D), lambda b: (b, 0)),
            pl.BlockSpec((FF, D), lambda b: (0, 0)),
            pl.BlockSpec((FF, D), lambda b: (0, 0)),
            pl.BlockSpec((D, FF), lambda b: (0, 0)),
        ],
        out_specs=pl.BlockSpec((R, D), lambda b: (b, 0)),
        out_shape=jax.ShapeDtypeStruct((T, D), jnp.float32),
        scratch_shapes=[
            pltpu.VMEM((FF, D), jnp.bfloat16),
            pltpu.VMEM((FF, D), jnp.bfloat16),
            pltpu.VMEM((D, FF), jnp.bfloat16),
        ],
        compiler_params=pltpu.CompilerParams(
            dimension_semantics=("arbitrary",)),
    )(xt, sw_gate, sw_up, sw_down)


def _ffn_kernel(xsb_ref, eid_ref, ob_ref, xs_ref, wg_ref, wu_ref, wd_ref,
                out_ref, bg_ref, bu_ref, bd_ref):
    del xsb_ref, ob_ref
    t = pl.program_id(0)
    new_expert = jnp.logical_or(t == 0, eid_ref[t] != eid_ref[t - 1])

    @pl.when(new_expert)
    def _():
        bg_ref[...] = wg_ref[0].astype(jnp.bfloat16)
        bu_ref[...] = wu_ref[0].astype(jnp.bfloat16)
        bd_ref[...] = wd_ref[0].astype(jnp.bfloat16)

    xb = xs_ref[...].astype(jnp.bfloat16)
    g = lax.dot_general(xb, bg_ref[...], (((1,), (1,)), ((), ())),
                        preferred_element_type=jnp.float32)
    u = lax.dot_general(xb, bu_ref[...], (((1,), (1,)), ((), ())),
                        preferred_element_type=jnp.float32)
    h = (g * lax.logistic(g) * u).astype(jnp.bfloat16)
    out_ref[...] = lax.dot_general(h, bd_ref[...], (((1,), (1,)), ((), ())),
                                   preferred_element_type=jnp.float32)


def _gmm(xs, w_gate, w_up, w_down, xs_blk, w_eid, out_blk):
    grid_spec = pltpu.PrefetchScalarGridSpec(
        num_scalar_prefetch=3,
        grid=(NBR,),
        in_specs=[
            pl.BlockSpec((R, D), lambda t, xsb, eid, ob: (xsb[t], 0)),
            pl.BlockSpec((1, FF, D), lambda t, xsb, eid, ob: (eid[t], 0, 0)),
            pl.BlockSpec((1, FF, D), lambda t, xsb, eid, ob: (eid[t], 0, 0)),
            pl.BlockSpec((1, D, FF), lambda t, xsb, eid, ob: (eid[t], 0, 0)),
        ],
        out_specs=pl.BlockSpec((R, D), lambda t, xsb, eid, ob: (ob[t], 0)),
        scratch_shapes=[
            pltpu.VMEM((FF, D), jnp.bfloat16),
            pltpu.VMEM((FF, D), jnp.bfloat16),
            pltpu.VMEM((D, FF), jnp.bfloat16),
        ],
    )
    return pl.pallas_call(
        _ffn_kernel,
        grid_spec=grid_spec,
        out_shape=jax.ShapeDtypeStruct((ROWS, D), jnp.float32),
        compiler_params=pltpu.CompilerParams(
            dimension_semantics=("arbitrary",)),
    )(xs_blk, w_eid, out_blk, xs, w_gate, w_up, w_down)


def _collect_kernel(out_hbm, pos_hbm, g_hbm, idx_v, rows_v, sem):
    wid = lax.axis_index("s") * _NC + lax.axis_index("c")
    base = wid * (2 * T // _NW)
    for c in range(2):
        g = base + c * 64
        pltpu.sync_copy(pos_hbm.at[pl.ds(g, 64)], idx_v.at[c])
        pltpu.async_copy(out_hbm.at[idx_v.at[c]], rows_v, sem).wait()
        pltpu.sync_copy(rows_v, g_hbm.at[pl.ds(g, 64)])


def _collect(out, pos):
    mesh = plsc.VectorSubcoreMesh(core_axis_name="c", subcore_axis_name="s")
    return pl.kernel(
        _collect_kernel,
        out_type=jax.ShapeDtypeStruct((2 * T, D), jnp.float32),
        mesh=mesh,
        scratch_types=[
            pltpu.VMEM((2, 64), jnp.int32),
            pltpu.VMEM((64, D), jnp.float32),
            pltpu.SemaphoreType.DMA,
        ],
    )(out, pos)


def _combine_kernel(g0_ref, g1_ref, sh_ref, w_ref, y_ref):
    b = pl.program_id(0)
    w0 = w_ref[pl.ds(b * R, R), 0:1]
    w1 = w_ref[pl.ds(b * R, R), 1:2]
    y_ref[...] = w0 * g0_ref[...] + w1 * g1_ref[...] + sh_ref[...]


def _combine(gath, out_sh, tw):
    return pl.pallas_call(
        _combine_kernel,
        grid=(T // R,),
        in_specs=[
            pl.BlockSpec((R, D), lambda b: (b, 0)),
            pl.BlockSpec((R, D), lambda b: (b + T // R, 0)),
            pl.BlockSpec((R, D), lambda b: (b, 0)),
            pl.BlockSpec((T, 2), lambda b: (0, 0)),
        ],
        out_specs=pl.BlockSpec((R, D), lambda b: (b, 0)),
        out_shape=jax.ShapeDtypeStruct((T, D), jnp.float32),
    )(gath, gath, out_sh, tw)


def kernel(x, gate_w, w_gate, w_up, w_down, sw_gate, sw_up, sw_down):
    b, s, h = x.shape
    xt = x.reshape(-1, h)

    pos, tw, cnt = _gate(xt, gate_w)

    c = cnt[0]
    nb = (c + R - 1) // R
    starts = jnp.cumsum(nb)
    nbr = starts[E - 1]
    ti = jnp.arange(NBR, dtype=jnp.int32)
    eid_r = jnp.minimum(
        jnp.sum((starts[None, :] <= ti[:, None]).astype(jnp.int32), axis=1),
        E - 1)
    xs_blk = jnp.minimum(ti, nbr - 1)
    out_blk = jnp.where(ti < nbr, ti, NB - 1)

    xs = _dispatch(xt, pos.reshape(2 * T))
    out_sh = _shared_ffn(xt, sw_gate, sw_up, sw_down)
    out = _gmm(xs, w_gate, w_up, w_down, xs_blk, eid_r, out_blk)
    gath = _collect(out, pos.reshape(2 * T))
    y = _combine(gath, out_sh, tw)
    return y.reshape(b, s, h)
